# baseline (speedup 1.0000x reference)
import functools

import numpy as np
import jax
import jax.numpy as jnp
from jax import lax
from jax.experimental import pallas as pl
from jax.experimental.pallas import tpu as pltpu
from jax.experimental.pallas import tpu_sc as plsc

B, L = 4096, 50
N = B * L
D_P = 96
D_PAD = 128
D_MODEL = 384

NW = 32
PER_W = N // NW
CH = 128
NCH = PER_W // CH

TB = 1024


def _sc_gather2(Ep, Eb, idx_p, idx_b):
    mesh = plsc.VectorSubcoreMesh(core_axis_name="c", subcore_axis_name="s")

    @functools.partial(
        pl.kernel,
        mesh=mesh,
        out_type=(
            jax.ShapeDtypeStruct((N, D_PAD), jnp.float32),
            jax.ShapeDtypeStruct((N, D_PAD), jnp.float32),
        ),
        scratch_types=[
            pltpu.VMEM((NCH, CH), jnp.int32),
            pltpu.VMEM((NCH, CH), jnp.int32),
            pltpu.VMEM((CH, D_PAD), jnp.float32),
            pltpu.VMEM((CH, D_PAD), jnp.float32),
            pltpu.SemaphoreType.DMA,
        ],
        compiler_params=pltpu.CompilerParams(use_tc_tiling_on_sc=True),
    )
    def k(ep_hbm, eb_hbm, ip_hbm, ib_hbm, op_hbm, ob_hbm,
          ipv, ibv, bufp, bufb, sem):
        cid = lax.axis_index("c")
        sid = lax.axis_index("s")
        wid = sid * 2 + cid
        base = wid * PER_W
        pltpu.sync_copy(ip_hbm.at[wid], ipv)
        pltpu.sync_copy(ib_hbm.at[wid], ibv)

        def body(j, carry):
            gp = pltpu.make_async_copy(ep_hbm.at[ipv.at[j]], bufp, sem)
            gb = pltpu.make_async_copy(eb_hbm.at[ibv.at[j]], bufb, sem)
            gp.start()
            gb.start()
            gp.wait()
            gb.wait()
            row0 = base + j * CH
            pltpu.sync_copy(bufp, op_hbm.at[pl.ds(row0, CH)])
            pltpu.sync_copy(bufb, ob_hbm.at[pl.ds(row0, CH)])
            return carry

        lax.fori_loop(0, NCH, body, 0)

    return k(Ep, Eb, idx_p, idx_b)


_INV_SQRT2 = np.float32(1.0 / np.sqrt(2.0))


def _tc_body(gp_r, gb_r, cp_r, sid_r, w1p_r, w1b_r, w1c_r, tsm_r, w2_r, b2_r,
             out_r):
    x = jnp.dot(gp_r[...], w1p_r[...], preferred_element_type=jnp.float32)
    x = x + jnp.dot(gb_r[...], w1b_r[...], preferred_element_type=jnp.float32)
    x = x + jnp.dot(cp_r[...], w1c_r[...], preferred_element_type=jnp.float32)
    oh = (lax.broadcasted_iota(jnp.int32, (TB, 64), 1) == sid_r[...]
          ).astype(jnp.float32)
    x = x + jnp.dot(oh, tsm_r[...], preferred_element_type=jnp.float32)
    x = 0.5 * x * (1.0 + lax.erf(x * _INV_SQRT2))
    out_r[...] = (jnp.dot(x, w2_r[...], preferred_element_type=jnp.float32)
                  + b2_r[...])


def _tc_mlp(gp, gb, cp, sidx, W1p, W1b, W1c, Tsm, W2, b2):
    grid = (N // TB,)
    return pl.pallas_call(
        _tc_body,
        grid=grid,
        in_specs=[
            pl.BlockSpec((TB, D_PAD), lambda i: (i, 0)),
            pl.BlockSpec((TB, D_PAD), lambda i: (i, 0)),
            pl.BlockSpec((TB, 21), lambda i: (i, 0)),
            pl.BlockSpec((TB, 1), lambda i: (i, 0)),
            pl.BlockSpec((D_PAD, D_MODEL), lambda i: (0, 0)),
            pl.BlockSpec((D_PAD, D_MODEL), lambda i: (0, 0)),
            pl.BlockSpec((21, D_MODEL), lambda i: (0, 0)),
            pl.BlockSpec((64, D_MODEL), lambda i: (0, 0)),
            pl.BlockSpec((D_MODEL, D_MODEL), lambda i: (0, 0)),
            pl.BlockSpec((1, D_MODEL), lambda i: (0, 0)),
        ],
        out_specs=pl.BlockSpec((TB, D_MODEL), lambda i: (i, 0)),
        out_shape=jax.ShapeDtypeStruct((N, D_MODEL), jnp.float32),
    )(gp, gb, cp, sidx, W1p, W1b, W1c, Tsm, W2, b2)


def kernel(pitcher_id, batter_id, p_throws_id, stand_id, inning_topbot_id,
           cont, profile, E_pitcher, E_batter, E_pthrows, E_stand, E_topbot,
           W1, b1, W2, b2):
    pid = pitcher_id.astype(jnp.int32).reshape(NW, NCH, CH)
    bid = batter_id.astype(jnp.int32).reshape(NW, NCH, CH)
    ep128 = jnp.pad(E_pitcher, ((0, 0), (0, D_PAD - D_P)))
    eb128 = jnp.pad(E_batter, ((0, 0), (0, D_PAD - D_P)))
    gp, gb = _sc_gather2(ep128, eb128, pid, bid)

    sidx = (p_throws_id.astype(jnp.int32) * 16
            + stand_id.astype(jnp.int32) * 4
            + inning_topbot_id.astype(jnp.int32)).reshape(N, 1)
    cp = jnp.concatenate([cont.reshape(N, 12), profile.reshape(N, 9)], axis=1)

    Tsm = (jnp.dot(E_pthrows, W1[192:196])[:, None, None, :]
           + jnp.dot(E_stand, W1[196:200])[None, :, None, :]
           + jnp.dot(E_topbot, W1[200:204])[None, None, :, :]
           + b1[None, None, None, :]).reshape(64, D_MODEL)

    w1p = jnp.pad(W1[0:96], ((0, D_PAD - D_P), (0, 0)))
    w1b = jnp.pad(W1[96:192], ((0, D_PAD - D_P), (0, 0)))
    out = _tc_mlp(gp, gb, cp, sidx,
                  w1p, w1b, W1[204:225], Tsm, W2, b2[None, :])
    return out.reshape(B, L, D_MODEL)


# speedup vs baseline: 1.5770x; 1.5770x over previous
import functools

import numpy as np
import jax
import jax.numpy as jnp
from jax import lax
from jax.experimental import pallas as pl
from jax.experimental.pallas import tpu as pltpu
from jax.experimental.pallas import tpu_sc as plsc

B, L = 4096, 50
N = B * L
V = 100000
D_P = 96
D_MODEL = 384

NW = 32
PER_W = N // NW
CH = 128
NCH = PER_W // CH

MB = 2000
TB = 1024


def _table_matmul_body(e_r, w_r, out_r):
    out_r[...] = jnp.dot(e_r[...], w_r[...], preferred_element_type=jnp.float32)


def _tc_table_matmul(E, W):
    return pl.pallas_call(
        _table_matmul_body,
        grid=(V // MB,),
        in_specs=[
            pl.BlockSpec((MB, D_P), lambda i: (i, 0)),
            pl.BlockSpec((D_P, D_MODEL), lambda i: (0, 0)),
        ],
        out_specs=pl.BlockSpec((MB, D_MODEL), lambda i: (i, 0)),
        out_shape=jax.ShapeDtypeStruct((V, D_MODEL), jnp.float32),
    )(E, W)


def _sc_gather_add(Ap, Ab, idx_p, idx_b):
    mesh = plsc.VectorSubcoreMesh(core_axis_name="c", subcore_axis_name="s")

    @functools.partial(
        pl.kernel,
        mesh=mesh,
        out_type=jax.ShapeDtypeStruct((N, D_MODEL), jnp.float32),
        scratch_types=[
            pltpu.VMEM((NCH, CH), jnp.int32),
            pltpu.VMEM((NCH, CH), jnp.int32),
            pltpu.VMEM((CH, D_MODEL), jnp.float32),
            pltpu.SemaphoreType.DMA,
        ],
    )
    def k(ap_hbm, ab_hbm, ip_hbm, ib_hbm, out_hbm, ipv, ibv, buf, sem):
        cid = lax.axis_index("c")
        sid = lax.axis_index("s")
        wid = sid * 2 + cid
        base = wid * PER_W
        pltpu.sync_copy(ip_hbm.at[wid], ipv)
        pltpu.sync_copy(ib_hbm.at[wid], ibv)

        def body(j, carry):
            pltpu.async_copy(ap_hbm.at[ipv.at[j]], buf, sem).wait()
            pltpu.async_copy(ab_hbm.at[ibv.at[j]], buf, sem, add=True).wait()
            pltpu.sync_copy(buf, out_hbm.at[pl.ds(base + j * CH, CH)])
            return carry

        lax.fori_loop(0, NCH, body, 0)

    return k(Ap, Ab, idx_p, idx_b)


_INV_SQRT2 = np.float32(1.0 / np.sqrt(2.0))


def _mlp_body(z1_r, cp_r, sid_r, w1c_r, tsm_r, w2_r, b2_r, out_r):
    x = z1_r[...] + jnp.dot(cp_r[...], w1c_r[...],
                            preferred_element_type=jnp.float32)
    oh = (lax.broadcasted_iota(jnp.int32, (TB, 64), 1) == sid_r[...]
          ).astype(jnp.float32)
    x = x + jnp.dot(oh, tsm_r[...], preferred_element_type=jnp.float32)
    x = 0.5 * x * (1.0 + lax.erf(x * _INV_SQRT2))
    out_r[...] = (jnp.dot(x, w2_r[...], preferred_element_type=jnp.float32)
                  + b2_r[...])


def _tc_mlp(z1, cp, sidx, W1c, Tsm, W2, b2):
    return pl.pallas_call(
        _mlp_body,
        grid=(N // TB,),
        in_specs=[
            pl.BlockSpec((TB, D_MODEL), lambda i: (i, 0)),
            pl.BlockSpec((TB, 21), lambda i: (i, 0)),
            pl.BlockSpec((TB, 1), lambda i: (i, 0)),
            pl.BlockSpec((21, D_MODEL), lambda i: (0, 0)),
            pl.BlockSpec((64, D_MODEL), lambda i: (0, 0)),
            pl.BlockSpec((D_MODEL, D_MODEL), lambda i: (0, 0)),
            pl.BlockSpec((1, D_MODEL), lambda i: (0, 0)),
        ],
        out_specs=pl.BlockSpec((TB, D_MODEL), lambda i: (i, 0)),
        out_shape=jax.ShapeDtypeStruct((N, D_MODEL), jnp.float32),
    )(z1, cp, sidx, W1c, Tsm, W2, b2)


def kernel(pitcher_id, batter_id, p_throws_id, stand_id, inning_topbot_id,
           cont, profile, E_pitcher, E_batter, E_pthrows, E_stand, E_topbot,
           W1, b1, W2, b2):
    pid = pitcher_id.astype(jnp.int32).T.reshape(NW, NCH, CH)
    bid = batter_id.astype(jnp.int32).T.reshape(NW, NCH, CH)

    Ap = _tc_table_matmul(E_pitcher, W1[0:96])
    Ab = _tc_table_matmul(E_batter, W1[96:192])
    z1 = _sc_gather_add(Ap, Ab, pid, bid)

    sidx = (p_throws_id.astype(jnp.int32) * 16
            + stand_id.astype(jnp.int32) * 4
            + inning_topbot_id.astype(jnp.int32)).T.reshape(N, 1)
    cp = jnp.concatenate([cont, profile], axis=-1).transpose(1, 0, 2).reshape(N, 21)

    Tsm = (jnp.dot(E_pthrows, W1[192:196])[:, None, None, :]
           + jnp.dot(E_stand, W1[196:200])[None, :, None, :]
           + jnp.dot(E_topbot, W1[200:204])[None, None, :, :]
           + b1[None, None, None, :]).reshape(64, D_MODEL)

    out = _tc_mlp(z1, cp, sidx, W1[204:225], Tsm, W2, b2[None, :])
    return out.reshape(L, B, D_MODEL).transpose(1, 0, 2)
